# baseline (speedup 1.0000x reference)
import functools

import jax
import jax.numpy as jnp
from jax import lax
from jax.experimental import pallas as pl
from jax.experimental.pallas import tpu as pltpu
from jax.experimental.pallas import tpu_sc as plsc

N_ROWS = 10000
B = 4096
HW = 121
F = 64
CD = 4
CDP = 16

NC = 2
NS = 16
NW = NC * NS
CPW = B // NW
FB = 4
SPW = F // FB
WORK = HW * SPW
TRIPS = 31
JBLK = 16


def _sc_coords(coords_hbm, idx_hbm, lab_hbm, idx_v, cbuf, csem):
    wid = lax.axis_index("s") * NC + lax.axis_index("c")
    pltpu.sync_copy(idx_hbm.at[pl.ds(wid * CPW, CPW)], idx_v)
    pltpu.async_copy(coords_hbm.at[idx_v], cbuf, csem).wait()
    pltpu.sync_copy(cbuf, lab_hbm.at[pl.ds(wid * CPW, CPW)])


def _sc_gather(tab_hbm, idx_hbm, out_hbm,
               idx_v, s0, s1, o0, o1, ls0, ls1, os0, os1):
    wid = lax.axis_index("s") * NC + lax.axis_index("c")

    pltpu.sync_copy(idx_hbm, idx_v)

    sbufs = (s0, s1)
    obufs = (o0, o1)
    lsems = (ls0, ls1)
    osems = (os0, os1)

    def cr(t):
        u = wid + t * NW
        return u, u // SPW, (u % SPW) * FB

    for k in (0, 1):
        _, c, r = cr(k)
        pltpu.async_copy(tab_hbm.at[c, pl.ds(r, FB)], sbufs[k], lsems[k])

    def pair(t2, carry):
        for k in (0, 1):
            t = 2 * t2 + k
            u, c, r = cr(t)

            @pl.when(u < WORK)
            def _(k=k, t=t, c=c, r=r):
                pltpu.make_async_copy(
                    tab_hbm.at[c, pl.ds(r, FB)], sbufs[k], lsems[k]).wait()

                @pl.when(t2 > 0)
                def _():
                    pltpu.make_async_copy(
                        obufs[k], out_hbm.at[c, pl.ds(r, FB)], osems[k]).wait()

                def jblock(jb4, carry2):
                    vals = []
                    for q in range(4):
                        jb = jb4 * 4 + q
                        cols = idx_v[pl.ds(jb * JBLK, JBLK)]
                        for f in range(FB):
                            rowv = jnp.full((JBLK,), f, jnp.int32)
                            vals.append(
                                (f, jb, plsc.load_gather(sbufs[k],
                                                         [rowv, cols])))
                    for f, jb, v in vals:
                        obufs[k][f, pl.ds(jb * JBLK, JBLK)] = v
                    return carry2

                lax.fori_loop(0, B // JBLK // 4, jblock, 0)

                pltpu.async_copy(obufs[k], out_hbm.at[c, pl.ds(r, FB)],
                                 osems[k])

                u2, c2, r2 = cr(t + 2)

                @pl.when(u2 < WORK)
                def _():
                    pltpu.async_copy(tab_hbm.at[c2, pl.ds(r2, FB)],
                                     sbufs[k], lsems[k])

        return carry

    lax.fori_loop(0, TRIPS, pair, 0)

    for k in (0, 1):
        pltpu.make_async_copy(
            obufs[k], out_hbm.at[0, pl.ds(0, FB)], osems[k]).wait()


@jax.jit
def _run(tab3, indices, coords_p):
    mesh = plsc.VectorSubcoreMesh(core_axis_name="c", subcore_axis_name="s")
    k = pl.kernel(
        _sc_gather,
        out_type=jax.ShapeDtypeStruct((HW, F, B), jnp.float32),
        mesh=mesh,
        scratch_types=[
            pltpu.VMEM((B,), jnp.int32),
            pltpu.VMEM((FB, N_ROWS), jnp.float32),
            pltpu.VMEM((FB, N_ROWS), jnp.float32),
            pltpu.VMEM((FB, B), jnp.float32),
            pltpu.VMEM((FB, B), jnp.float32),
            pltpu.SemaphoreType.DMA,
            pltpu.SemaphoreType.DMA,
            pltpu.SemaphoreType.DMA,
            pltpu.SemaphoreType.DMA,
        ],
        compiler_params=pltpu.CompilerParams(
            use_tc_tiling_on_sc=True, needs_layout_passes=False),
    )
    kc = pl.kernel(
        _sc_coords,
        out_type=jax.ShapeDtypeStruct((B, CDP), jnp.float32),
        mesh=plsc.VectorSubcoreMesh(core_axis_name="c", subcore_axis_name="s"),
        scratch_types=[
            pltpu.VMEM((CPW,), jnp.int32),
            pltpu.VMEM((CPW, CDP), jnp.float32),
            pltpu.SemaphoreType.DMA,
        ],
        compiler_params=pltpu.CompilerParams(use_tc_tiling_on_sc=False),
    )
    return k(tab3, indices), kc(coords_p, indices)


_PERM_CACHE = []


def _perm():
    if not _PERM_CACHE:
        _PERM_CACHE.append(
            jax.random.permutation(jax.random.key(1), N_ROWS))
    return _PERM_CACHE[0]


def kernel(num_samples, slices, coords):
    n = slices.shape[0]
    indices = lax.dynamic_slice_in_dim(
        _perm(), num_samples - B, B).astype(jnp.int32)
    tab3 = jnp.transpose(slices, (2, 3, 1, 0)).reshape(HW, F, n)
    coords_p = jnp.pad(coords, ((0, 0), (0, CDP - CD)))
    out3, labels_p = _run(tab3, indices, coords_p)
    samples = jnp.transpose(
        out3.reshape(11, 11, F, B), (3, 2, 0, 1))
    return (samples, labels_p[:, :CD])


# speedup vs baseline: 1.1688x; 1.1688x over previous
import functools

import jax
import jax.numpy as jnp
from jax import lax
from jax.experimental import pallas as pl
from jax.experimental.pallas import tpu as pltpu
from jax.experimental.pallas import tpu_sc as plsc

N_ROWS = 10000
B = 4096
HW = 121
F = 64
CD = 4
CDP = 16

NC = 2
NS = 16
NW = NC * NS
CPW = B // NW
FB = 4
SPW = F // FB
WORK = HW * SPW
TRIPS = 31
JBLK = 16


def _sc_coords(coords_hbm, idx_hbm, lab_hbm, idx_v, cbuf, csem):
    wid = lax.axis_index("s") * NC + lax.axis_index("c")
    pltpu.sync_copy(idx_hbm.at[pl.ds(wid * CPW, CPW)], idx_v)
    pltpu.async_copy(coords_hbm.at[idx_v], cbuf, csem).wait()
    pltpu.sync_copy(cbuf, lab_hbm.at[pl.ds(wid * CPW, CPW)])


def _sc_gather(tab_hbm, idx_hbm, out_hbm,
               idx_v, s0, s1, o0, o1, ls0, ls1, os0, os1):
    wid = lax.axis_index("s") * NC + lax.axis_index("c")

    pltpu.sync_copy(idx_hbm, idx_v)

    sbufs = (s0, s1)
    obufs = (o0, o1)
    lsems = (ls0, ls1)
    osems = (os0, os1)

    def cr(t):
        u = wid + t * NW
        return u, u // SPW, (u % SPW) * FB

    for k in (0, 1):
        _, c, r = cr(k)
        pltpu.async_copy(tab_hbm.at[c, pl.ds(r, FB)], sbufs[k], lsems[k])

    def pair(t2, carry):
        for k in (0, 1):
            t = 2 * t2 + k
            u, c, r = cr(t)

            @pl.when(u < WORK)
            def _(k=k, t=t, c=c, r=r):
                pltpu.make_async_copy(
                    tab_hbm.at[c, pl.ds(r, FB)], sbufs[k], lsems[k]).wait()

                @pl.when(t2 > 0)
                def _():
                    pltpu.make_async_copy(
                        obufs[k], out_hbm.at[c, pl.ds(r, FB)], osems[k]).wait()

                def jblock(jb4, carry2):
                    vals = []
                    for q in range(4):
                        jb = jb4 * 4 + q
                        cols = idx_v[pl.ds(jb * JBLK, JBLK)]
                        for f in range(FB):
                            rowv = jnp.full((JBLK,), f, jnp.int32)
                            vals.append(
                                (f, jb, plsc.load_gather(sbufs[k],
                                                         [rowv, cols])))
                    for f, jb, v in vals:
                        obufs[k][f, pl.ds(jb * JBLK, JBLK)] = v
                    return carry2

                lax.fori_loop(0, B // JBLK // 4, jblock, 0)

                pltpu.async_copy(obufs[k], out_hbm.at[c, pl.ds(r, FB)],
                                 osems[k])

                u2, c2, r2 = cr(t + 2)

                @pl.when(u2 < WORK)
                def _():
                    pltpu.async_copy(tab_hbm.at[c2, pl.ds(r2, FB)],
                                     sbufs[k], lsems[k])

        return carry

    lax.fori_loop(0, TRIPS, pair, 0)

    for k in (0, 1):
        pltpu.make_async_copy(
            obufs[k], out_hbm.at[0, pl.ds(0, FB)], osems[k]).wait()


@jax.jit
def _run(tab3, indices, coords_p):
    mesh = plsc.VectorSubcoreMesh(core_axis_name="c", subcore_axis_name="s")
    k = pl.kernel(
        _sc_gather,
        out_type=jax.ShapeDtypeStruct((HW, F, B), jnp.float32),
        mesh=mesh,
        scratch_types=[
            pltpu.VMEM((B,), jnp.int32),
            pltpu.VMEM((FB, N_ROWS), jnp.float32),
            pltpu.VMEM((FB, N_ROWS), jnp.float32),
            pltpu.VMEM((FB, B), jnp.float32),
            pltpu.VMEM((FB, B), jnp.float32),
            pltpu.SemaphoreType.DMA,
            pltpu.SemaphoreType.DMA,
            pltpu.SemaphoreType.DMA,
            pltpu.SemaphoreType.DMA,
        ],
        compiler_params=pltpu.CompilerParams(
            use_tc_tiling_on_sc=True, needs_layout_passes=False),
    )
    kc = pl.kernel(
        _sc_coords,
        out_type=jax.ShapeDtypeStruct((B, CDP), jnp.float32),
        mesh=plsc.VectorSubcoreMesh(core_axis_name="c", subcore_axis_name="s"),
        scratch_types=[
            pltpu.VMEM((CPW,), jnp.int32),
            pltpu.VMEM((CPW, CDP), jnp.float32),
            pltpu.SemaphoreType.DMA,
        ],
        compiler_params=pltpu.CompilerParams(use_tc_tiling_on_sc=False),
    )
    return k(tab3, indices), kc(coords_p, indices)


_PERM_CACHE = []


def _perm():
    if not _PERM_CACHE:
        with jax.ensure_compile_time_eval():
            _PERM_CACHE.append(
                jax.random.permutation(jax.random.key(1), N_ROWS))
    return _PERM_CACHE[0]


def kernel(num_samples, slices, coords):
    n = slices.shape[0]
    indices = lax.dynamic_slice_in_dim(
        _perm(), num_samples - B, B).astype(jnp.int32)
    tab3 = jnp.transpose(slices, (2, 3, 1, 0)).reshape(HW, F, n)
    coords_p = jnp.pad(coords, ((0, 0), (0, CDP - CD)))
    out3, labels_p = _run(tab3, indices, coords_p)
    samples = jnp.transpose(
        out3.reshape(11, 11, F, B), (3, 2, 0, 1))
    return (samples, labels_p[:, :CD])


# speedup vs baseline: 1.1868x; 1.0154x over previous
import functools

import jax
import jax.numpy as jnp
from jax import lax
from jax.experimental import pallas as pl
from jax.experimental.pallas import tpu as pltpu
from jax.experimental.pallas import tpu_sc as plsc

N_ROWS = 10000
B = 4096
HW = 121
F = 64
CD = 4
CDP = 128

NC = 2
NS = 16
NW = NC * NS
CPW = B // NW
FB = 4
SPW = F // FB
WORK = HW * SPW
TRIPS = 31
JBLK = 16


def _sc_gather(tab_hbm, idx_hbm, coords_hbm, out_hbm, lab_hbm,
               idx_v, s0, s1, o0, o1, cbuf, ls0, ls1, os0, os1, csem):
    wid = lax.axis_index("s") * NC + lax.axis_index("c")

    pltpu.sync_copy(idx_hbm, idx_v)

    cbase = wid * CPW
    pltpu.async_copy(
        coords_hbm.at[idx_v.at[pl.ds(cbase, CPW // 2)]], cbuf, csem)

    sbufs = (s0, s1)
    obufs = (o0, o1)
    lsems = (ls0, ls1)
    osems = (os0, os1)

    def cr(t):
        u = wid + t * NW
        return u, u // SPW, (u % SPW) * FB

    for k in (0, 1):
        _, c, r = cr(k)
        pltpu.async_copy(tab_hbm.at[c, pl.ds(r, FB)], sbufs[k], lsems[k])

    def pair(t2, carry):
        for k in (0, 1):
            t = 2 * t2 + k
            u, c, r = cr(t)

            @pl.when(u < WORK)
            def _(k=k, t=t, c=c, r=r):
                pltpu.make_async_copy(
                    tab_hbm.at[c, pl.ds(r, FB)], sbufs[k], lsems[k]).wait()

                @pl.when(t2 > 0)
                def _():
                    pltpu.make_async_copy(
                        obufs[k], out_hbm.at[c, pl.ds(r, FB)], osems[k]).wait()

                def jblock(jb4, carry2):
                    vals = []
                    for q in range(4):
                        jb = jb4 * 4 + q
                        cols = idx_v[pl.ds(jb * JBLK, JBLK)]
                        for f in range(FB):
                            rowv = jnp.full((JBLK,), f, jnp.int32)
                            vals.append(
                                (f, jb, plsc.load_gather(sbufs[k],
                                                         [rowv, cols])))
                    for f, jb, v in vals:
                        obufs[k][f, pl.ds(jb * JBLK, JBLK)] = v
                    return carry2

                lax.fori_loop(0, B // JBLK // 4, jblock, 0)

                pltpu.async_copy(obufs[k], out_hbm.at[c, pl.ds(r, FB)],
                                 osems[k])

                u2, c2, r2 = cr(t + 2)

                @pl.when(u2 < WORK)
                def _():
                    pltpu.async_copy(tab_hbm.at[c2, pl.ds(r2, FB)],
                                     sbufs[k], lsems[k])

        return carry

    lax.fori_loop(0, TRIPS, pair, 0)

    for k in (0, 1):
        pltpu.make_async_copy(
            obufs[k], out_hbm.at[0, pl.ds(0, FB)], osems[k]).wait()

    pltpu.make_async_copy(
        coords_hbm.at[idx_v.at[pl.ds(cbase, CPW // 2)]], cbuf, csem).wait()
    pltpu.sync_copy(cbuf, lab_hbm.at[pl.ds(cbase, CPW // 2)])
    pltpu.async_copy(
        coords_hbm.at[idx_v.at[pl.ds(cbase + CPW // 2, CPW // 2)]],
        cbuf, csem).wait()
    pltpu.sync_copy(cbuf, lab_hbm.at[pl.ds(cbase + CPW // 2, CPW // 2)])


@jax.jit
def _run(tab3, indices, coords_p):
    mesh = plsc.VectorSubcoreMesh(core_axis_name="c", subcore_axis_name="s")
    k = pl.kernel(
        _sc_gather,
        out_type=(
            jax.ShapeDtypeStruct((HW, F, B), jnp.float32),
            jax.ShapeDtypeStruct((B, CDP), jnp.float32),
        ),
        mesh=mesh,
        scratch_types=[
            pltpu.VMEM((B,), jnp.int32),
            pltpu.VMEM((FB, N_ROWS), jnp.float32),
            pltpu.VMEM((FB, N_ROWS), jnp.float32),
            pltpu.VMEM((FB, B), jnp.float32),
            pltpu.VMEM((FB, B), jnp.float32),
            pltpu.VMEM((CPW // 2, CDP), jnp.float32),
            pltpu.SemaphoreType.DMA,
            pltpu.SemaphoreType.DMA,
            pltpu.SemaphoreType.DMA,
            pltpu.SemaphoreType.DMA,
            pltpu.SemaphoreType.DMA,
        ],
        compiler_params=pltpu.CompilerParams(
            use_tc_tiling_on_sc=True, needs_layout_passes=False),
    )
    return k(tab3, indices, coords_p)


_PERM_CACHE = []


def _perm():
    if not _PERM_CACHE:
        with jax.ensure_compile_time_eval():
            _PERM_CACHE.append(
                jax.random.permutation(jax.random.key(1), N_ROWS))
    return _PERM_CACHE[0]


def kernel(num_samples, slices, coords):
    n = slices.shape[0]
    indices = lax.dynamic_slice_in_dim(
        _perm(), num_samples - B, B).astype(jnp.int32)
    tab3 = jnp.transpose(slices, (2, 3, 1, 0)).reshape(HW, F, n)
    coords_p = jnp.pad(coords, ((0, 0), (0, CDP - CD)))
    out3, labels_p = _run(tab3, indices, coords_p)
    samples = jnp.transpose(
        out3.reshape(11, 11, F, B), (3, 2, 0, 1))
    return (samples, labels_p[:, :CD])


# speedup vs baseline: 1.1892x; 1.0021x over previous
import jax
import jax.numpy as jnp
from jax import lax
from jax.experimental import pallas as pl
from jax.experimental.pallas import tpu as pltpu
from jax.experimental.pallas import tpu_sc as plsc

N_ROWS = 10000
B = 4096
HW = 121
F = 64
CD = 4
CDP = 128

NC = 2
NS = 16
NW = NC * NS
CPW = B // NW
FB = 4
SPW = F // FB
WORK = HW * SPW
TRIPS = 31
JBLK = 16


def _sc_gather(tab_hbm, idx_hbm, coords_hbm, out_hbm, lab_hbm,
               idx_v, s0, s1, o0, o1, cbuf, ls0, ls1, os0, os1, csem):
    wid = lax.axis_index("s") * NC + lax.axis_index("c")

    pltpu.sync_copy(idx_hbm, idx_v)

    cbase = wid * CPW
    pltpu.async_copy(
        coords_hbm.at[idx_v.at[pl.ds(cbase, CPW // 2)]], cbuf, csem)

    sbufs = (s0, s1)
    obufs = (o0, o1)
    lsems = (ls0, ls1)
    osems = (os0, os1)

    def cr(t):
        u = wid + t * NW
        return u, u // SPW, (u % SPW) * FB

    for k in (0, 1):
        _, c, r = cr(k)
        pltpu.async_copy(tab_hbm.at[c, pl.ds(r, FB)], sbufs[k], lsems[k])

    def pair(t2, carry):
        for k in (0, 1):
            t = 2 * t2 + k
            u, c, r = cr(t)

            @pl.when(u < WORK)
            def _(k=k, t=t, c=c, r=r):
                pltpu.make_async_copy(
                    tab_hbm.at[c, pl.ds(r, FB)], sbufs[k], lsems[k]).wait()

                @pl.when(t2 > 0)
                def _():
                    pltpu.make_async_copy(
                        obufs[k], out_hbm.at[c, pl.ds(r, FB)], osems[k]).wait()

                def jblock(jb4, carry2):
                    vals = []
                    for q in range(4):
                        jb = jb4 * 4 + q
                        cols = idx_v[pl.ds(jb * JBLK, JBLK)]
                        for f in range(FB):
                            rowv = jnp.full((JBLK,), f, jnp.int32)
                            vals.append(
                                (f, jb, plsc.load_gather(sbufs[k],
                                                         [rowv, cols])))
                    for f, jb, v in vals:
                        obufs[k][f, pl.ds(jb * JBLK, JBLK)] = v
                    return carry2

                lax.fori_loop(0, B // JBLK // 4, jblock, 0)

                pltpu.async_copy(obufs[k], out_hbm.at[c, pl.ds(r, FB)],
                                 osems[k])

                u2, c2, r2 = cr(t + 2)

                @pl.when(u2 < WORK)
                def _():
                    pltpu.async_copy(tab_hbm.at[c2, pl.ds(r2, FB)],
                                     sbufs[k], lsems[k])

        return carry

    lax.fori_loop(0, TRIPS, pair, 0)

    for k in (0, 1):
        pltpu.make_async_copy(
            obufs[k], out_hbm.at[0, pl.ds(0, FB)], osems[k]).wait()

    pltpu.make_async_copy(
        coords_hbm.at[idx_v.at[pl.ds(cbase, CPW // 2)]], cbuf, csem).wait()
    pltpu.sync_copy(cbuf, lab_hbm.at[pl.ds(cbase, CPW // 2)])
    pltpu.async_copy(
        coords_hbm.at[idx_v.at[pl.ds(cbase + CPW // 2, CPW // 2)]],
        cbuf, csem).wait()
    pltpu.sync_copy(cbuf, lab_hbm.at[pl.ds(cbase + CPW // 2, CPW // 2)])


@jax.jit
def _run(tab3, indices, coords_p):
    mesh = plsc.VectorSubcoreMesh(core_axis_name="c", subcore_axis_name="s")
    k = pl.kernel(
        _sc_gather,
        out_type=(
            jax.ShapeDtypeStruct((HW, F, B), jnp.float32),
            jax.ShapeDtypeStruct((B, CDP), jnp.float32),
        ),
        mesh=mesh,
        scratch_types=[
            pltpu.VMEM((B,), jnp.int32),
            pltpu.VMEM((FB, N_ROWS), jnp.float32),
            pltpu.VMEM((FB, N_ROWS), jnp.float32),
            pltpu.VMEM((FB, B), jnp.float32),
            pltpu.VMEM((FB, B), jnp.float32),
            pltpu.VMEM((CPW // 2, CDP), jnp.float32),
            pltpu.SemaphoreType.DMA,
            pltpu.SemaphoreType.DMA,
            pltpu.SemaphoreType.DMA,
            pltpu.SemaphoreType.DMA,
            pltpu.SemaphoreType.DMA,
        ],
        compiler_params=pltpu.CompilerParams(
            use_tc_tiling_on_sc=True, needs_layout_passes=False),
    )
    return k(tab3, indices, coords_p)


_PERM_CACHE = []


def _perm():
    if not _PERM_CACHE:
        with jax.ensure_compile_time_eval():
            _PERM_CACHE.append(
                jax.random.permutation(jax.random.key(1), N_ROWS))
    return _PERM_CACHE[0]


def kernel(num_samples, slices, coords):
    n = slices.shape[0]
    indices = lax.dynamic_slice_in_dim(
        _perm(), num_samples - B, B).astype(jnp.int32)
    tab3 = jnp.transpose(slices, (2, 3, 1, 0)).reshape(HW, F, n)
    coords_p = jnp.pad(coords, ((0, 0), (0, CDP - CD)))
    out3, labels_p = _run(tab3, indices, coords_p)
    samples = jnp.transpose(
        out3.reshape(11, 11, F, B), (3, 2, 0, 1))
    return (samples, labels_p[:, :CD])
